# Initial kernel scaffold; baseline (speedup 1.0000x reference)
import jax
import jax.numpy as jnp
from jax.experimental import pallas as pl


def kernel(nodes, edges, neighbor_idxs, params):
    raise NotImplementedError("write your pallas kernel here")


# speedup vs baseline: 12.3546x; 12.3546x over previous
import functools

import jax
import jax.numpy as jnp
from jax import lax
from jax.experimental import pallas as pl
from jax.experimental.pallas import tpu as pltpu
from jax.experimental.pallas import tpu_sc as plsc

_F32 = jnp.float32

_NC, _NS = 2, 16
_NW = _NC * _NS
_CH = 200
_K = 2
_NB = 2


def _sc_mesh():
    return plsc.VectorSubcoreMesh(core_axis_name="c", subcore_axis_name="s",
                                  num_cores=_NC, num_subcores=_NS)


def _proj_body(x_ref, wr_ref, ws_ref, out_ref):
    x = x_ref[...]
    v = x.shape[0]
    out_ref[pl.ds(0, v), :] = jnp.dot(x, wr_ref[...], preferred_element_type=_F32)
    out_ref[pl.ds(v, v), :] = jnp.dot(x, ws_ref[...], preferred_element_type=_F32)


def _proj_tables(nodes_flat, w0r, w0s):
    v, d = nodes_flat.shape
    return pl.pallas_call(
        _proj_body,
        out_shape=jax.ShapeDtypeStruct((2 * v, d), _F32),
    )(nodes_flat, w0r, w0s)


def _edge_body(x_ref, gr_ref, gs_ref, w0_ref, b0_ref, w1_ref, b1_ref,
               w2_ref, b2_ref, g_ref, be_ref, eo_ref, en_ref):
    x = x_ref[...]
    h = jnp.dot(x, w0_ref[...], preferred_element_type=_F32)
    h = jnp.maximum(h + gr_ref[...] + gs_ref[...] + b0_ref[...], 0.0)
    h = jnp.maximum(jnp.dot(h, w1_ref[...], preferred_element_type=_F32) + b1_ref[...], 0.0)
    o = jnp.dot(h, w2_ref[...], preferred_element_type=_F32) + b2_ref[...]
    m = jnp.mean(o, axis=-1, keepdims=True)
    dd = o - m
    var = jnp.mean(dd * dd, axis=-1, keepdims=True)
    o = dd * lax.rsqrt(var + 1e-5) * g_ref[...] + be_ref[...]
    eo_ref[...] = o
    en_ref[...] = x + o


def _edge_mlp(edges, row_off, ek, gr, gs, w0e, b0, w1, b1, w2, b2, g, be):
    d = edges.shape[1]
    bm = 2000
    grid = ek // bm
    boff = row_off // bm
    erow = pl.BlockSpec((bm, d), lambda i: (boff + i, 0))
    row = pl.BlockSpec((bm, d), lambda i: (i, 0))
    full = pl.BlockSpec((d, d), lambda i: (0, 0))
    vec = pl.BlockSpec((1, d), lambda i: (0, 0))
    return pl.pallas_call(
        _edge_body,
        grid=(grid,),
        in_specs=[erow, row, row, full, vec, full, vec, full, vec, vec, vec],
        out_specs=(row, row),
        out_shape=(jax.ShapeDtypeStruct((ek, d), _F32),
                   jax.ShapeDtypeStruct((ek, d), _F32)),
    )(edges, gr, gs, w0e, b0, w1, b1, w2, b2, g, be)


def _node_body(*refs):
    (x_ref, wa_ref, wn_ref, b0_ref, w1_ref, b1_ref,
     w2_ref, b2_ref, g_ref, bn_ref, out_ref) = refs[-11:]
    p_refs = refs[:-11]
    agg = p_refs[0][0] + p_refs[0][1]
    for pr in p_refs[1:]:
        agg = agg + pr[0] + pr[1]
    x = x_ref[...]
    h = (jnp.dot(agg, wa_ref[...], preferred_element_type=_F32)
         + jnp.dot(x, wn_ref[...], preferred_element_type=_F32) + b0_ref[...])
    h = jnp.maximum(h, 0.0)
    h = jnp.maximum(jnp.dot(h, w1_ref[...], preferred_element_type=_F32) + b1_ref[...], 0.0)
    o = jnp.dot(h, w2_ref[...], preferred_element_type=_F32) + b2_ref[...]
    m = jnp.mean(o, axis=-1, keepdims=True)
    dd = o - m
    var = jnp.mean(dd * dd, axis=-1, keepdims=True)
    o = dd * lax.rsqrt(var + 1e-5) * g_ref[...] + bn_ref[...]
    out_ref[...] = x + o


def _node_mlp(partials, nodes_flat, wa, wn, b0, w1, b1, w2, b2, g, bn):
    v, d = nodes_flat.shape
    bm = v // 8
    grid = v // bm
    prow = pl.BlockSpec((2, bm, d), lambda i: (0, i, 0))
    row = pl.BlockSpec((bm, d), lambda i: (i, 0))
    full = pl.BlockSpec((d, d), lambda i: (0, 0))
    vec = pl.BlockSpec((1, d), lambda i: (0, 0))
    return pl.pallas_call(
        _node_body,
        grid=(grid,),
        in_specs=[prow] * len(partials) + [row, full, full, vec, full, vec, full, vec, vec, vec],
        out_specs=row,
        out_shape=jax.ShapeDtypeStruct((v, d), _F32),
    )(*partials, nodes_flat, wa, wn, b0, w1, b1, w2, b2, g, bn)


def _gather(table, idx3d, ek):
    d = table.shape[1]
    rpt = ek // _NW
    n_ch = rpt // _CH
    t_tot = 2 * n_ch
    n_grp = -(-t_tot // _NB)

    @functools.partial(
        pl.kernel, mesh=_sc_mesh(),
        out_type=(jax.ShapeDtypeStruct((ek, d), _F32),
                  jax.ShapeDtypeStruct((ek, d), _F32)),
        scratch_types=[pltpu.VMEM((_CH,), jnp.int32) for _ in range(_NB)]
        + [pltpu.VMEM((_CH, d), _F32)] * _NB
        + [pltpu.SemaphoreType.DMA] * (3 * _NB),
    )
    def gather_k(table_hbm, idx_hbm, gr_hbm, gs_hbm, *rest):
        ibufs = rest[:_NB]
        bufs = rest[_NB:2 * _NB]
        isems = rest[2 * _NB:3 * _NB]
        gsems = rest[3 * _NB:4 * _NB]
        wsems = rest[4 * _NB:]
        wid = lax.axis_index("s") * _NC + lax.axis_index("c")
        base = wid * rpt

        def start_idx(j, b):
            pltpu.make_async_copy(idx_hbm.at[wid, j], ibufs[b], isems[b]).start()

        def start_gather(b):
            pltpu.make_async_copy(table_hbm.at[ibufs[b]], bufs[b], gsems[b]).start()

        def wait_idx(b):
            pltpu.make_async_copy(idx_hbm.at[wid, 0], ibufs[b], isems[b]).wait()

        def wait_gather(b):
            pltpu.make_async_copy(table_hbm.at[ibufs[b]], bufs[b], gsems[b]).wait()

        def wait_wb(b):
            pltpu.make_async_copy(bufs[b], gr_hbm.at[pl.ds(base, _CH)], wsems[b]).wait()

        for b in range(_NB):
            start_idx(b, b)
        for b in range(_NB):
            wait_idx(b)
            start_gather(b)

        def group(g, carry):
            for b in range(_NB):
                j = g * _NB + b
                jn = j + _NB

                @pl.when(j < t_tot)
                def _(j=j, b=b):
                    wait_gather(b)

                    @pl.when(j < n_ch)
                    def _():
                        pltpu.make_async_copy(
                            bufs[b], gr_hbm.at[pl.ds(base + j * _CH, _CH)], wsems[b]).start()

                    @pl.when(j >= n_ch)
                    def _():
                        pltpu.make_async_copy(
                            bufs[b], gs_hbm.at[pl.ds(base + (j - n_ch) * _CH, _CH)], wsems[b]).start()

                @pl.when(jn < t_tot)
                def _(jn=jn, b=b):
                    start_idx(jn, b)

            for b in range(_NB):
                jn = g * _NB + b + _NB

                @pl.when(jn < t_tot)
                def _(jn=jn, b=b):
                    wait_wb(b)
                    wait_idx(b)
                    start_gather(b)

            return carry

        lax.fori_loop(0, n_grp, group, 0)
        for b in range(_NB):
            wait_wb(b)

    return gather_k(table, idx3d)


def _scatter_add(eo, idx, zeros):
    e, d = eo.shape
    v = zeros.shape[0]
    rpt = e // _NW
    n_ch = rpt // _CH
    vpt = v // _NS

    @functools.partial(
        pl.kernel, mesh=_sc_mesh(),
        out_type=jax.ShapeDtypeStruct((_NC, v, d), _F32),
        scratch_types=[pltpu.VMEM((_CH,), jnp.int32),
                       pltpu.VMEM((_CH, d), _F32),
                       pltpu.VMEM_SHARED((v, d), _F32)],
    )
    def scatter_k(eo_hbm, idx_hbm, zeros_hbm, out_hbm, idx_v, rows_v, acc_sh):
        cid = lax.axis_index("c")
        sid = lax.axis_index("s")
        wid = sid * _NC + cid
        base = wid * rpt
        pltpu.sync_copy(zeros_hbm.at[pl.ds(sid * vpt, vpt)],
                        acc_sh.at[pl.ds(sid * vpt, vpt)])
        plsc.subcore_barrier()

        def body(i, carry):
            off = base + i * _CH
            pltpu.sync_copy(idx_hbm.at[pl.ds(off, _CH)], idx_v)
            pltpu.sync_copy(eo_hbm.at[pl.ds(off, _CH)], rows_v)
            pltpu.sync_copy(rows_v, acc_sh.at[idx_v], add=True)
            return carry

        lax.fori_loop(0, n_ch, body, 0)
        plsc.subcore_barrier()
        pltpu.sync_copy(acc_sh.at[pl.ds(sid * vpt, vpt)],
                        out_hbm.at[cid, pl.ds(sid * vpt, vpt)])

    return scatter_k(eo, idx, zeros)


def kernel(nodes, edges, neighbor_idxs, params):
    b, n, d = nodes.shape
    e = edges.shape[0]
    v = b * n
    vp = -(-v // 128) * 128
    ni = neighbor_idxs.astype(jnp.int32)
    flat_r = ni[:, 0] * n + ni[:, 1]
    flat_s = ni[:, 0] * n + ni[:, 2] + vp
    zeros_h = jnp.zeros((vp, d), _F32)
    x_nodes = jnp.zeros((vp, d), _F32).at[:v].set(nodes.reshape(v, d))
    x_edges = edges

    ek = e // _K
    n_ch_g = (ek // _NW) // _CH
    n_ch_s = (ek // _NS) // _CH
    idx_g, idx_s = [], []
    for k in range(_K):
        fr = lax.dynamic_slice_in_dim(flat_r, k * ek, ek)
        fs = lax.dynamic_slice_in_dim(flat_s, k * ek, ek)
        idx_g.append(jnp.concatenate([fr.reshape(_NW, n_ch_g, _CH),
                                      fs.reshape(_NW, n_ch_g, _CH)], axis=1))
        idx_s.append(fr)

    for layer in params:
        (w0, b0), (w1, b1), (w2, b2) = layer["edge_mlp"]
        g_e, be = layer["edge_ln"]
        w0e, w0r, w0s = w0[:d], w0[d:2 * d], w0[2 * d:]
        table = _proj_tables(x_nodes, w0r, w0s)
        partials, en_chunks = [], []
        for k in range(_K):
            gr, gs = _gather(table, idx_g[k], ek)
            if isinstance(x_edges, list):
                e_arr, off = x_edges[k], 0
            else:
                e_arr, off = x_edges, k * ek
            eo2, en = _edge_mlp(e_arr, off, ek, gr, gs, w0e, b0.reshape(1, d),
                                w1, b1.reshape(1, d), w2, b2.reshape(1, d),
                                g_e.reshape(1, d), be.reshape(1, d))
            en_chunks.append(en)
            partials.append(_scatter_add(eo2, idx_s[k], zeros_h))
        x_edges = en_chunks
        (wn0, nb0), (wn1, nb1), (wn2, nb2) = layer["node_mlp"]
        g_n, bn = layer["node_ln"]
        wa, wn = wn0[:d], wn0[d:]
        x_nodes = _node_mlp(partials, x_nodes, wa, wn, nb0.reshape(1, d), wn1,
                            nb1.reshape(1, d), wn2, nb2.reshape(1, d),
                            g_n.reshape(1, d), bn.reshape(1, d))

    return (x_nodes[:v].reshape(b, n, d),
            jnp.concatenate(x_edges, axis=0), neighbor_idxs)
